# baseline (speedup 1.0000x reference)
import functools

import jax
import jax.numpy as jnp
from jax.experimental import pallas as pl
from jax.experimental.pallas import tpu as pltpu

E = 16
TOPK = 2
BLOCK_T = 1664


def _moe_body(x_ref, gw_ref, ewm1_ref, o_ref):
    x = x_ref[...]
    logits = jax.lax.dot_general(
        x, gw_ref[...], (((1,), (1,)), ((), ())),
        preferred_element_type=jnp.float32)

    v0 = jnp.max(logits, axis=1, keepdims=True)
    mask0 = logits == v0
    rest = jnp.where(mask0, -jnp.inf, logits)
    v1 = jnp.max(rest, axis=1, keepdims=True)
    mask1 = rest == v1

    w0 = 1.0 / (1.0 + jnp.exp(v1 - v0))
    w = jnp.where(mask0, w0, 0.0) + jnp.where(mask1, 1.0 - w0, 0.0)

    corr = jax.lax.dot_general(
        w.astype(jnp.bfloat16), ewm1_ref[...], (((1,), (0,)), ((), ())),
        preferred_element_type=jnp.float32)
    o_ref[...] = x + x * corr


@functools.partial(jax.jit, static_argnames=())
def kernel(norm_data, gate_w, expert_w):
    T, D = norm_data.shape
    ewm1_b = (expert_w - 1.0).astype(jnp.bfloat16)
    grid = (pl.cdiv(T, BLOCK_T),)
    return pl.pallas_call(
        _moe_body,
        grid=grid,
        in_specs=[
            pl.BlockSpec((BLOCK_T, D), lambda i: (i, 0)),
            pl.BlockSpec((E, D), lambda i: (0, 0)),
            pl.BlockSpec((E, D), lambda i: (0, 0)),
        ],
        out_specs=pl.BlockSpec((BLOCK_T, D), lambda i: (i, 0)),
        out_shape=jax.ShapeDtypeStruct((T, D), norm_data.dtype),
        compiler_params=pltpu.CompilerParams(
            dimension_semantics=("parallel",),
        ),
    )(norm_data, gate_w, ewm1_b)
